# baseline (speedup 1.0000x reference)
import jax
import jax.numpy as jnp
from jax.experimental import pallas as pl
from jax.experimental.pallas import tpu as pltpu


def _gcn_body(x_ref, w_ref, adj_a_ref, adj_b_ref, sumrow_ref, y_ref, b_ref,
              out_ref, support_ref):
    @pl.when(pl.program_id(0) == 0)
    def _():
        support_ref[...] = jnp.dot(
            x_ref[...], w_ref[...], preferred_element_type=jnp.float32)

    support = support_ref[...]
    agg_a = jnp.dot(adj_a_ref[...], support, preferred_element_type=jnp.float32)
    agg_b = jnp.dot(adj_b_ref[...], support, preferred_element_type=jnp.float32)
    agg = jnp.concatenate([agg_a, agg_b], axis=0)
    out_ref[...] = jnp.maximum(
        agg / sumrow_ref[...] + y_ref[...] + b_ref[...], 0.0)


def kernel(x, y, adj, adj_sumrow, W, b):
    N, d_in = x.shape
    d_out = W.shape[1]
    BR = 512
    b2 = b.reshape(1, d_out)
    return pl.pallas_call(
        _gcn_body,
        grid=(N // (2 * BR),),
        in_specs=[
            pl.BlockSpec((N, d_in), lambda i: (0, 0)),
            pl.BlockSpec((d_in, d_out), lambda i: (0, 0)),
            pl.BlockSpec((BR, N), lambda i: (2 * i, 0)),
            pl.BlockSpec((BR, N), lambda i: (2 * i + 1, 0)),
            pl.BlockSpec((2 * BR, 1), lambda i: (i, 0)),
            pl.BlockSpec((2 * BR, d_out), lambda i: (i, 0)),
            pl.BlockSpec((1, d_out), lambda i: (0, 0)),
        ],
        out_specs=pl.BlockSpec((2 * BR, d_out), lambda i: (i, 0)),
        out_shape=jax.ShapeDtypeStruct((N, d_out), jnp.float32),
        scratch_shapes=[pltpu.VMEM((N, d_out), jnp.float32)],
    )(x, W, adj, adj, adj_sumrow, y, b2)


# speedup vs baseline: 1.0806x; 1.0806x over previous
import jax
import jax.numpy as jnp
from jax.experimental import pallas as pl
from jax.experimental.pallas import tpu as pltpu

_BR = 256
_K = 4


def _gcn_body(x_ref, w_ref, adj_ref, sumrow_ref, y_ref, b_ref, out_ref,
              abuf, ybuf, obuf, support, asem, ysem, osem):
    n, d_in = x_ref.shape
    nb = n // _BR

    def adj_copy(j, slot):
        return pltpu.make_async_copy(
            adj_ref.at[pl.ds(j * _BR, _BR), :], abuf.at[slot], asem.at[slot])

    def y_copy(j, slot):
        return pltpu.make_async_copy(
            y_ref.at[pl.ds(j * _BR, _BR), :], ybuf.at[slot], ysem.at[slot])

    def out_copy(j, slot):
        return pltpu.make_async_copy(
            obuf.at[slot], out_ref.at[pl.ds(j * _BR, _BR), :], osem.at[slot])

    for k in range(_K):
        adj_copy(k, k).start()
        y_copy(k, k).start()

    support[...] = jnp.dot(
        x_ref[...], w_ref[...], preferred_element_type=jnp.float32)

    for j in range(nb):
        s = j % _K
        o = j % 2
        adj_copy(j, s).wait()
        y_copy(j, s).wait()
        agg = jnp.dot(
            abuf[s], support[...], preferred_element_type=jnp.float32)
        if j >= 2:
            out_copy(j - 2, o).wait()
        obuf[o] = jnp.maximum(
            agg / sumrow_ref[pl.ds(j * _BR, _BR), :] + ybuf[s] + b_ref[...],
            0.0)
        out_copy(j, o).start()
        if j + _K < nb:
            adj_copy(j + _K, s).start()
            y_copy(j + _K, s).start()

    for j in (nb - 2, nb - 1):
        out_copy(j, j % 2).wait()


def kernel(x, y, adj, adj_sumrow, W, b):
    N, d_in = x.shape
    d_out = W.shape[1]
    b2 = b.reshape(1, d_out)
    return pl.pallas_call(
        _gcn_body,
        in_specs=[
            pl.BlockSpec(memory_space=pltpu.VMEM),
            pl.BlockSpec(memory_space=pltpu.VMEM),
            pl.BlockSpec(memory_space=pltpu.HBM),
            pl.BlockSpec(memory_space=pltpu.VMEM),
            pl.BlockSpec(memory_space=pltpu.HBM),
            pl.BlockSpec(memory_space=pltpu.VMEM),
        ],
        out_specs=pl.BlockSpec(memory_space=pltpu.HBM),
        out_shape=jax.ShapeDtypeStruct((N, d_out), jnp.float32),
        scratch_shapes=[
            pltpu.VMEM((_K, _BR, N), jnp.float32),
            pltpu.VMEM((_K, _BR, d_out), jnp.float32),
            pltpu.VMEM((2, _BR, d_out), jnp.float32),
            pltpu.VMEM((N, d_out), jnp.float32),
            pltpu.SemaphoreType.DMA((_K,)),
            pltpu.SemaphoreType.DMA((_K,)),
            pltpu.SemaphoreType.DMA((2,)),
        ],
    )(x, W, adj, adj_sumrow, y, b2)
